# baseline (speedup 1.0000x reference)
import functools

import jax
import jax.numpy as jnp
from jax import lax
from jax.experimental import pallas as pl
from jax.experimental.pallas import tpu as pltpu
from jax.experimental.pallas import tpu_sc as plsc

T, D, F, E = 32, 2048, 2048, 16
BF = 512
NF = F // BF
NSUB = 16
TPW = T // NSUB


def _sc_router(x_hbm, rk_hbm, fw_hbm, order_hbm,
               rk_v, x_v, row_v, orow_v, mat_v, act_sh):
    cid = lax.axis_index("c")
    sid = lax.axis_index("s")

    @pl.when(cid == 0)
    def _():
        pltpu.sync_copy(rk_hbm, rk_v)
        iota = lax.iota(jnp.int32, E)
        act = jnp.zeros((E,), jnp.float32)
        for j in range(TPW):
            t = sid * TPW + j
            pltpu.sync_copy(x_hbm.at[t], x_v)

            def body(c, acc):
                base = c * 16
                xc = x_v[pl.ds(base, 16)]
                for k in range(16):
                    acc = acc + xc[k] * rk_v[pl.ds((base + k) * E, E)]
                return acc

            logits = lax.fori_loop(0, D // 16, body,
                                   jnp.zeros((E,), jnp.float32))
            m = jnp.max(logits)
            ex = jnp.exp(logits - m)
            p = ex / jnp.full((E,), jnp.sum(ex))
            m1 = jnp.max(p)
            a1 = jnp.min(jnp.where(p == m1, iota, E))
            pm = jnp.where(iota == a1, -1.0, p)
            m2 = jnp.max(pm)
            a2 = jnp.min(jnp.where(pm == m2, iota, E))
            e2 = jnp.exp(jnp.full((E,), m2 - m1, jnp.float32))
            inv = 1.0 / (1.0 + e2)
            sel1 = iota == a1
            sel2 = iota == a2
            row_v[...] = (jnp.where(sel1, inv, 0.0)
                          + jnp.where(sel2, e2 * inv, 0.0))
            sel = sel1 | sel2
            pltpu.sync_copy(row_v, fw_hbm.at[t])
            act = jnp.maximum(act, jnp.where(sel, 1.0, 0.0))
        row_v[...] = act
        pltpu.sync_copy(row_v, act_sh.at[sid])
        plsc.subcore_barrier()

        @pl.when(sid == 0)
        def _():
            pltpu.sync_copy(act_sh, mat_v)
            acc = mat_v[0, :]
            for i in range(1, NSUB):
                acc = jnp.maximum(acc, mat_v[i, :])
            prev = plsc.cummax(jnp.where(acc > 0.0, iota, -1))
            first_a = jnp.min(jnp.where(acc > 0.0, iota, E))
            orow_v[...] = jnp.where(prev >= 0, prev, first_a)
            pltpu.sync_copy(orow_v, order_hbm)


_sc_router_call = functools.partial(
    pl.kernel,
    out_type=(jax.ShapeDtypeStruct((T, E), jnp.float32),
              jax.ShapeDtypeStruct((E,), jnp.int32)),
    mesh=plsc.VectorSubcoreMesh(core_axis_name="c", subcore_axis_name="s"),
    scratch_types=[
        pltpu.VMEM((D * E,), jnp.float32),
        pltpu.VMEM((D,), jnp.float32),
        pltpu.VMEM((E,), jnp.float32),
        pltpu.VMEM((E,), jnp.int32),
        pltpu.VMEM((NSUB, E), jnp.float32),
        pltpu.VMEM_SHARED((NSUB, E), jnp.float32),
    ],
    compiler_params=pltpu.CompilerParams(needs_layout_passes=False),
)(_sc_router)


def _moe_kernel(order_ref, x_ref, fw_ref, wg_ref, wu_ref, wd_ref, out_ref):
    f = pl.program_id(0)
    s = pl.program_id(1)

    @pl.when((f == 0) & (s == 0))
    def _():
        out_ref[...] = jnp.zeros_like(out_ref)

    @pl.when(order_ref[s] == s)
    def _():
        x = x_ref[...]
        gate = jnp.dot(x, wg_ref[0], preferred_element_type=jnp.float32)
        up = jnp.dot(x, wu_ref[0], preferred_element_type=jnp.float32)
        hidden = (gate * jax.lax.logistic(gate)) * up
        contrib = jnp.dot(hidden, wd_ref[0], preferred_element_type=jnp.float32)
        iota = jax.lax.broadcasted_iota(jnp.int32, (T, E), 1)
        scale = jnp.sum(jnp.where(iota == s, fw_ref[...], 0.0), axis=1,
                        keepdims=True)
        out_ref[...] += contrib * scale


def kernel(x, router_kernel, w_gate, w_up, w_down):
    fw, order = _sc_router_call(x, router_kernel.reshape(D * E))
    return pl.pallas_call(
        _moe_kernel,
        grid_spec=pltpu.PrefetchScalarGridSpec(
            num_scalar_prefetch=1,
            grid=(NF, E),
            in_specs=[
                pl.BlockSpec((T, D), lambda f, s, order: (0, 0)),
                pl.BlockSpec((T, E), lambda f, s, order: (0, 0)),
                pl.BlockSpec((1, D, BF), lambda f, s, order: (order[s], 0, f)),
                pl.BlockSpec((1, D, BF), lambda f, s, order: (order[s], 0, f)),
                pl.BlockSpec((1, BF, D), lambda f, s, order: (order[s], f, 0)),
            ],
            out_specs=pl.BlockSpec((1, T, D), lambda f, s, order: (0, 0, 0)),
        ),
        out_shape=jax.ShapeDtypeStruct((1, T, D), jnp.float32),
        compiler_params=pltpu.CompilerParams(
            dimension_semantics=("arbitrary", "arbitrary")),
    )(order, x, fw, w_gate, w_up, w_down)[0]


# speedup vs baseline: 1.0316x; 1.0316x over previous
import jax
import jax.numpy as jnp
from jax.experimental import pallas as pl
from jax.experimental.pallas import tpu as pltpu

T, D, F, E = 32, 2048, 2048, 16
BF = 256
NF = F // BF


def _router_kernel(x_ref, rk_ref, fw_ref, order_ref):
    logits = jnp.dot(x_ref[...], rk_ref[...], preferred_element_type=jnp.float32)
    m = jnp.max(logits, axis=1, keepdims=True)
    ex = jnp.exp(logits - m)
    p = ex / jnp.sum(ex, axis=1, keepdims=True)
    iota = jax.lax.broadcasted_iota(jnp.int32, (T, E), 1)
    m1 = jnp.max(p, axis=1, keepdims=True)
    a1 = jnp.min(jnp.where(p == m1, iota, E), axis=1, keepdims=True)
    pm = jnp.where(iota == a1, -1.0, p)
    m2 = jnp.max(pm, axis=1, keepdims=True)
    a2 = jnp.min(jnp.where(pm == m2, iota, E), axis=1, keepdims=True)
    e2 = jnp.exp(m2 - m1)
    n1 = 1.0 / (1.0 + e2)
    n2 = e2 / (1.0 + e2)
    fw = jnp.where(iota == a1, n1, 0.0) + jnp.where(iota == a2, n2, 0.0)
    fw_ref[...] = fw
    act_row = jnp.sum(fw, axis=0, keepdims=True) > 0.0
    iota_l = jax.lax.broadcasted_iota(jnp.int32, (E, E), 1)
    iota_s = jax.lax.broadcasted_iota(jnp.int32, (E, E), 0)
    prev_col = jnp.max(jnp.where((iota_l <= iota_s) & act_row, iota_l, -1),
                       axis=1, keepdims=True)
    iota_row = jax.lax.broadcasted_iota(jnp.int32, (1, E), 1)
    first_a = jnp.min(jnp.where(act_row, iota_row, E), axis=1, keepdims=True)
    order_ref[...] = jnp.where(prev_col >= 0, prev_col, first_a)


def _moe_kernel(order_ref, x_ref, fw_ref, wg_ref, wu_ref, wd_ref, out_ref):
    f = pl.program_id(0)
    s = pl.program_id(1)

    @pl.when((f == 0) & (s == 0))
    def _():
        out_ref[...] = jnp.zeros_like(out_ref)

    @pl.when(order_ref[s] == s)
    def _():
        x = x_ref[...]
        gate = jnp.dot(x, wg_ref[0], preferred_element_type=jnp.float32)
        up = jnp.dot(x, wu_ref[0], preferred_element_type=jnp.float32)
        hidden = (gate * jax.lax.logistic(gate)) * up
        contrib = jnp.dot(hidden, wd_ref[0], preferred_element_type=jnp.float32)
        iota = jax.lax.broadcasted_iota(jnp.int32, (T, E), 1)
        scale = jnp.sum(jnp.where(iota == s, fw_ref[...], 0.0), axis=1,
                        keepdims=True)
        out_ref[...] += contrib * scale


def kernel(x, router_kernel, w_gate, w_up, w_down):
    fw, order = pl.pallas_call(
        _router_kernel,
        out_shape=(jax.ShapeDtypeStruct((T, E), jnp.float32),
                   jax.ShapeDtypeStruct((E, 1), jnp.int32)),
    )(x, router_kernel)
    order = order.reshape(E)
    return pl.pallas_call(
        _moe_kernel,
        grid_spec=pltpu.PrefetchScalarGridSpec(
            num_scalar_prefetch=1,
            grid=(NF, E),
            in_specs=[
                pl.BlockSpec((T, D), lambda f, s, order: (0, 0)),
                pl.BlockSpec((T, E), lambda f, s, order: (0, 0)),
                pl.BlockSpec((1, D, BF), lambda f, s, order: (order[s], 0, f)),
                pl.BlockSpec((1, D, BF), lambda f, s, order: (order[s], 0, f)),
                pl.BlockSpec((1, BF, D), lambda f, s, order: (order[s], f, 0)),
            ],
            out_specs=pl.BlockSpec((1, T, D), lambda f, s, order: (0, 0, 0)),
        ),
        out_shape=jax.ShapeDtypeStruct((1, T, D), jnp.float32),
        compiler_params=pltpu.CompilerParams(
            dimension_semantics=("arbitrary", "arbitrary")),
    )(order, x, fw, w_gate, w_up, w_down)[0]


# speedup vs baseline: 1.1157x; 1.0815x over previous
import jax
import jax.numpy as jnp
from jax.experimental import pallas as pl
from jax.experimental.pallas import tpu as pltpu

T, D, F, E = 32, 2048, 2048, 16
BF = 512
NF = F // BF


def _router_kernel(x_ref, rkt_ref, fw_ref, order_ref):
    x = x_ref[...]
    iota_te = jax.lax.broadcasted_iota(jnp.int32, (T, E), 1)
    logits = jnp.zeros((T, E), jnp.float32)
    for e in range(E):
        s = jnp.sum(x * rkt_ref[pl.ds(e, 1), :], axis=1, keepdims=True)
        logits = logits + jnp.where(iota_te == e, s, 0.0)
    m = jnp.max(logits, axis=1, keepdims=True)
    ex = jnp.exp(logits - m)
    p = ex / jnp.sum(ex, axis=1, keepdims=True)
    iota = jax.lax.broadcasted_iota(jnp.int32, (T, E), 1)
    m1 = jnp.max(p, axis=1, keepdims=True)
    a1 = jnp.min(jnp.where(p == m1, iota, E), axis=1, keepdims=True)
    pm = jnp.where(iota == a1, -1.0, p)
    m2 = jnp.max(pm, axis=1, keepdims=True)
    a2 = jnp.min(jnp.where(pm == m2, iota, E), axis=1, keepdims=True)
    e2 = jnp.exp(m2 - m1)
    n1 = 1.0 / (1.0 + e2)
    n2 = e2 / (1.0 + e2)
    fw = jnp.where(iota == a1, n1, 0.0) + jnp.where(iota == a2, n2, 0.0)
    fw_ref[...] = fw
    act_row = jnp.sum(fw, axis=0, keepdims=True) > 0.0
    iota_l = jax.lax.broadcasted_iota(jnp.int32, (E, E), 1)
    iota_s = jax.lax.broadcasted_iota(jnp.int32, (E, E), 0)
    prev_col = jnp.max(jnp.where((iota_l <= iota_s) & act_row, iota_l, -1),
                       axis=1, keepdims=True)
    iota_row = jax.lax.broadcasted_iota(jnp.int32, (1, E), 1)
    first_a = jnp.min(jnp.where(act_row, iota_row, E), axis=1, keepdims=True)
    order_ref[...] = jnp.where(prev_col >= 0, prev_col, first_a)


def _moe_kernel(order_ref, x_ref, fw_ref, wg_ref, wu_ref, wd_ref, out_ref):
    f = pl.program_id(0)
    s = pl.program_id(1)

    @pl.when((f == 0) & (s == 0))
    def _():
        out_ref[...] = jnp.zeros_like(out_ref)

    @pl.when(order_ref[s] == s)
    def _():
        x = x_ref[...]
        gate = jnp.dot(x, wg_ref[0], preferred_element_type=jnp.float32)
        up = jnp.dot(x, wu_ref[0], preferred_element_type=jnp.float32)
        hidden = (gate * jax.lax.logistic(gate)) * up
        contrib = jnp.dot(hidden, wd_ref[0], preferred_element_type=jnp.float32)
        iota = jax.lax.broadcasted_iota(jnp.int32, (T, E), 1)
        scale = jnp.sum(jnp.where(iota == s, fw_ref[...], 0.0), axis=1,
                        keepdims=True)
        out_ref[...] += contrib * scale


def kernel(x, router_kernel, w_gate, w_up, w_down):
    fw, order = pl.pallas_call(
        _router_kernel,
        out_shape=(jax.ShapeDtypeStruct((T, E), jnp.float32),
                   jax.ShapeDtypeStruct((E, 1), jnp.int32)),
    )(x, router_kernel.T)
    order = order.reshape(E)
    return pl.pallas_call(
        _moe_kernel,
        grid_spec=pltpu.PrefetchScalarGridSpec(
            num_scalar_prefetch=1,
            grid=(NF, E),
            in_specs=[
                pl.BlockSpec((T, D), lambda f, s, order: (0, 0)),
                pl.BlockSpec((T, E), lambda f, s, order: (0, 0)),
                pl.BlockSpec((1, D, BF), lambda f, s, order: (order[s], 0, f)),
                pl.BlockSpec((1, D, BF), lambda f, s, order: (order[s], 0, f)),
                pl.BlockSpec((1, BF, D), lambda f, s, order: (order[s], f, 0)),
            ],
            out_specs=pl.BlockSpec((1, T, D), lambda f, s, order: (0, 0, 0)),
        ),
        out_shape=jax.ShapeDtypeStruct((1, T, D), jnp.float32),
        compiler_params=pltpu.CompilerParams(
            dimension_semantics=("arbitrary", "arbitrary")),
    )(order, x, fw, w_gate, w_up, w_down)[0]
